# Initial kernel scaffold; baseline (speedup 1.0000x reference)
import jax
import jax.numpy as jnp
from jax.experimental import pallas as pl


def kernel(t, z, edge_index, W_cur, b_cur, W_nbr, b_nbr, W_out, b_out, W_g, b_g):
    raise NotImplementedError("write your pallas kernel here")


# speedup vs baseline: 40.9910x; 40.9910x over previous
import functools

import jax
import jax.numpy as jnp
from jax import lax
from jax.experimental import pallas as pl
from jax.experimental.pallas import tpu as pltpu
from jax.experimental.pallas import tpu_sc as plsc

S, N, E = 4, 10000, 320000
DIM_C, DIM_H, HID = 64, 64, 20
DIM_Z = DIM_C + DIM_H
ROW = S * HID

NC, NS = 2, 16
NT = NC * NS
CH = 128
EPT = 10240
EP = NT * EPT
CPT = EPT // CH
N_ACC = 10112
RPT = N_ACC // NS


def _celu(x):
    return jnp.where(x > 0, x, jnp.exp(jnp.minimum(x, 0.0)) - 1.0)


def _softplus(x):
    return jnp.maximum(x, 0.0) + jnp.log(1.0 + jnp.exp(-jnp.abs(x)))


def _vnbr_body(z_ref, wt_ref, b_ref, out_ref):
    w = wt_ref[...]
    b = b_ref[...]
    pieces = []
    for s in range(S):
        x = z_ref[s]
        v = _celu(jnp.dot(x, w, preferred_element_type=jnp.float32) + b)
        pieces.append(v)
    out_ref[...] = jnp.concatenate(pieces, axis=-1)


def _compute_vnbr(z, wnbr_t, b_nbr):
    bn = 1000
    return pl.pallas_call(
        _vnbr_body,
        grid=(N // bn,),
        in_specs=[
            pl.BlockSpec((S, bn, DIM_Z), lambda i: (0, i, 0)),
            pl.BlockSpec((DIM_Z, HID), lambda i: (0, 0)),
            pl.BlockSpec((1, HID), lambda i: (0, 0)),
        ],
        out_specs=pl.BlockSpec((bn, ROW), lambda i: (i, 0)),
        out_shape=jax.ShapeDtypeStruct((N, ROW), jnp.float32),
    )(z, wnbr_t, b_nbr)


def _sc_agg_body(vnbr_hbm, src_hbm, dst_hbm, zrow_hbm, out_hbm,
                 src_v, dst_v, rows_v, acc, sem):
    cid = lax.axis_index("c")
    sid = lax.axis_index("s")
    tid = cid * NS + sid

    pltpu.sync_copy(zrow_hbm, acc.at[pl.ds(sid * RPT, RPT)])
    plsc.subcore_barrier()

    def chunk(g, _):
        row = tid * CPT + g
        pltpu.sync_copy(src_hbm.at[row], src_v)
        pltpu.sync_copy(dst_hbm.at[row], dst_v)
        pltpu.async_copy(vnbr_hbm.at[src_v], rows_v, sem).wait()
        pltpu.sync_copy(rows_v, acc.at[dst_v], add=True)
        return ()

    lax.fori_loop(0, CPT, chunk, (), unroll=False)

    plsc.subcore_barrier()
    pltpu.sync_copy(acc.at[pl.ds(sid * RPT, RPT)],
                    out_hbm.at[cid, pl.ds(sid * RPT, RPT)])


def _sc_aggregate(vnbr_flat, srcp, dstp, zrow):
    mesh = plsc.VectorSubcoreMesh(core_axis_name="c", subcore_axis_name="s")
    fn = functools.partial(
        pl.kernel, mesh=mesh,
        compiler_params=pltpu.CompilerParams(use_tc_tiling_on_sc=False),
        out_type=jax.ShapeDtypeStruct((NC, N_ACC, ROW), jnp.float32),
        scratch_types=[
            pltpu.VMEM((CH,), jnp.int32),
            pltpu.VMEM((CH,), jnp.int32),
            pltpu.VMEM((CH, ROW), jnp.float32),
            pltpu.VMEM_SHARED((N_ACC, ROW), jnp.float32),
            pltpu.SemaphoreType.DMA,
        ],
    )(_sc_agg_body)
    return fn(vnbr_flat, srcp, dstp, zrow)


def _finish_body(z_ref, agg_ref, wcur_ref, bcur_ref, wo1_ref, wo2_ref,
                 bout_ref, wg_ref, bg_ref, out_ref):
    wcur = wcur_ref[...]
    bcur = bcur_ref[...]
    wo1 = wo1_ref[...]
    wo2 = wo2_ref[...]
    bout = bout_ref[...]
    wg = wg_ref[...]
    bg = bg_ref[...]
    for s in range(S):
        x = z_ref[s]
        c = x[:, :DIM_C]
        h = x[:, DIM_C:]
        vcur = _celu(jnp.dot(x, wcur, preferred_element_type=jnp.float32)
                     + bcur)
        a = (agg_ref[0, :, s * HID:(s + 1) * HID]
             + agg_ref[1, :, s * HID:(s + 1) * HID])
        dc0 = (jnp.dot(vcur, wo1, preferred_element_type=jnp.float32)
               + jnp.dot(a, wo2, preferred_element_type=jnp.float32)
               + bout)
        num = jnp.sum(dc0 * c, axis=-1, keepdims=True)
        den = jnp.sum(c * c, axis=-1, keepdims=True)
        dc = dc0 - (num / den) * c
        gx = jnp.dot(c, wg, preferred_element_type=jnp.float32) + bg
        dh = -_softplus(gx) * h
        out_ref[s] = jnp.concatenate([dc, dh], axis=-1)


def _finish(z, agg2, wcur_t, b_cur, wo1_t, wo2_t, b_out, wg_t, b_g):
    bn = 1000
    return pl.pallas_call(
        _finish_body,
        grid=(N // bn,),
        in_specs=[
            pl.BlockSpec((S, bn, DIM_Z), lambda i: (0, i, 0)),
            pl.BlockSpec((2, bn, ROW), lambda i: (0, i, 0)),
            pl.BlockSpec((DIM_Z, HID), lambda i: (0, 0)),
            pl.BlockSpec((1, HID), lambda i: (0, 0)),
            pl.BlockSpec((HID, DIM_C), lambda i: (0, 0)),
            pl.BlockSpec((HID, DIM_C), lambda i: (0, 0)),
            pl.BlockSpec((1, DIM_C), lambda i: (0, 0)),
            pl.BlockSpec((DIM_C, DIM_H), lambda i: (0, 0)),
            pl.BlockSpec((1, DIM_H), lambda i: (0, 0)),
        ],
        out_specs=pl.BlockSpec((S, bn, DIM_Z), lambda i: (0, i, 0)),
        out_shape=jax.ShapeDtypeStruct((S, N, DIM_Z), jnp.float32),
    )(z, agg2, wcur_t, b_cur, wo1_t, wo2_t, b_out, wg_t, b_g)


def kernel(t, z, edge_index, W_cur, b_cur, W_nbr, b_nbr, W_out, b_out, W_g, b_g):
    del t
    vnbr_flat = _compute_vnbr(z, W_nbr.T, b_nbr.reshape(1, HID))

    src = jnp.concatenate(
        [edge_index[0], jnp.zeros((EP - E,), jnp.int32)]).reshape(EP // CH, CH)
    dst = jnp.concatenate(
        [edge_index[1], jnp.full((EP - E,), N, jnp.int32)]).reshape(EP // CH, CH)
    zrow = jnp.zeros((RPT, ROW), jnp.float32)

    agg_pair = _sc_aggregate(vnbr_flat, src, dst, zrow)
    agg2 = agg_pair[:, :N, :]

    return _finish(z, agg2, W_cur.T, b_cur.reshape(1, HID),
                   W_out[:, :HID].T, W_out[:, HID:].T, b_out.reshape(1, DIM_C),
                   W_g.T, b_g.reshape(1, DIM_H))
